# Initial kernel scaffold; baseline (speedup 1.0000x reference)
import jax
import jax.numpy as jnp
from jax.experimental import pallas as pl


def kernel(feat, pos, W1, b1, W2, b2):
    raise NotImplementedError("write your pallas kernel here")


# speedup vs baseline: 7.4968x; 7.4968x over previous
import functools

import jax
import jax.numpy as jnp
from jax import lax
from jax.experimental import pallas as pl

N = 2048
NC = 512
K = 16
F = 128
H = 256


def _fps_kernel(pos_ref, out_ref):
    x = pos_ref[0, 0:1, :]
    y = pos_ref[0, 1:2, :]
    z = pos_ref[0, 2:3, :]
    iota = lax.broadcasted_iota(jnp.int32, (1, N), 1)
    out_ref[0, 0:1, :] = jnp.zeros((1, 1), jnp.int32)
    lx = x[:, 0:1]
    ly = y[:, 0:1]
    lz = z[:, 0:1]
    dmin0 = jnp.full((1, N), 1e10, jnp.float32)

    def step(i, carry):
        dmin, lx, ly, lz = carry
        d2 = (x - lx) ** 2 + (y - ly) ** 2 + (z - lz) ** 2
        dmin = jnp.minimum(dmin, d2)
        m = jnp.max(dmin, axis=1, keepdims=True)
        idx = jnp.min(jnp.where(dmin == m, iota, N), axis=1, keepdims=True)
        out_ref[0, pl.ds(i, 1), :] = idx
        sel = iota == idx
        lx = jnp.sum(jnp.where(sel, x, 0.0), axis=1, keepdims=True)
        ly = jnp.sum(jnp.where(sel, y, 0.0), axis=1, keepdims=True)
        lz = jnp.sum(jnp.where(sel, z, 0.0), axis=1, keepdims=True)
        return dmin, lx, ly, lz

    lax.fori_loop(1, NC, step, (dmin0, lx, ly, lz))


def _main_kernel(pos_r_ref, pos_c_ref, feat_ref, cent_ref,
                 w1f_ref, w1p_ref, b1_ref, w2_ref, b2_ref,
                 posq_ref, featq_ref):
    f32 = jnp.float32
    x = pos_r_ref[0, 0:1, :]
    y = pos_r_ref[0, 1:2, :]
    z = pos_r_ref[0, 2:3, :]
    xc = pos_c_ref[0, :, 0:1]
    yc = pos_c_ref[0, :, 1:2]
    zc = pos_c_ref[0, :, 2:3]
    feat = feat_ref[0]
    cent = cent_ref[0]

    P = (xc * w1p_ref[0:1, :] + yc * w1p_ref[1:2, :] + zc * w1p_ref[2:3, :])
    A = jnp.dot(feat, w1f_ref[...], preferred_element_type=f32) + P

    iota = lax.broadcasted_iota(jnp.int32, (NC, N), 1)
    ohc = (iota == cent)
    qx = jnp.sum(jnp.where(ohc, x, 0.0), axis=1, keepdims=True)
    qy = jnp.sum(jnp.where(ohc, y, 0.0), axis=1, keepdims=True)
    qz = jnp.sum(jnp.where(ohc, z, 0.0), axis=1, keepdims=True)
    qproj = jnp.dot(ohc.astype(f32), P, preferred_element_type=f32)
    cadd = b1_ref[...] - qproj

    D = (qx - x) ** 2 + (qy - y) ** 2 + (qz - z) ** 2

    acc = jnp.full((NC, H), -jnp.inf, f32)
    w2 = w2_ref[...]
    b2 = b2_ref[...]
    for _ in range(K):
        mn = jnp.min(D, axis=1, keepdims=True)
        idx = jnp.min(jnp.where(D == mn, iota, N), axis=1, keepdims=True)
        oh = (iota == idx)
        g = jnp.dot(oh.astype(f32), A, preferred_element_type=f32)
        h1 = jnp.maximum(g + cadd, 0.0)
        h2 = jnp.maximum(jnp.dot(h1, w2, preferred_element_type=f32) + b2, 0.0)
        acc = jnp.maximum(acc, h2)
        D = jnp.where(oh, jnp.inf, D)

    lane = lax.broadcasted_iota(jnp.int32, (NC, 128), 1)
    pq = jnp.where(lane == 0, qx, jnp.where(lane == 1, qy,
                                            jnp.where(lane == 2, qz, 0.0)))
    posq_ref[0] = pq
    featq_ref[0] = acc


@jax.jit
def kernel(feat, pos, W1, b1, W2, b2):
    b = feat.shape[0]
    f32 = jnp.float32
    pos_r = jnp.transpose(pos, (0, 2, 1))

    cent = pl.pallas_call(
        _fps_kernel,
        grid=(b,),
        in_specs=[pl.BlockSpec((1, 3, N), lambda i: (i, 0, 0))],
        out_specs=pl.BlockSpec((1, NC, 1), lambda i: (i, 0, 0)),
        out_shape=jax.ShapeDtypeStruct((b, NC, 1), jnp.int32),
    )(pos_r)

    w1f = W1[:F]
    w1p = jnp.zeros((8, H), f32).at[:3].set(W1[F:])
    b1r = b1.reshape(1, H)
    b2r = b2.reshape(1, H)

    posq, featq = pl.pallas_call(
        _main_kernel,
        grid=(b,),
        in_specs=[
            pl.BlockSpec((1, 3, N), lambda i: (i, 0, 0)),
            pl.BlockSpec((1, N, 3), lambda i: (i, 0, 0)),
            pl.BlockSpec((1, N, F), lambda i: (i, 0, 0)),
            pl.BlockSpec((1, NC, 1), lambda i: (i, 0, 0)),
            pl.BlockSpec((F, H), lambda i: (0, 0)),
            pl.BlockSpec((8, H), lambda i: (0, 0)),
            pl.BlockSpec((1, H), lambda i: (0, 0)),
            pl.BlockSpec((H, H), lambda i: (0, 0)),
            pl.BlockSpec((1, H), lambda i: (0, 0)),
        ],
        out_specs=[
            pl.BlockSpec((1, NC, 128), lambda i: (i, 0, 0)),
            pl.BlockSpec((1, NC, H), lambda i: (i, 0, 0)),
        ],
        out_shape=[
            jax.ShapeDtypeStruct((b, NC, 128), f32),
            jax.ShapeDtypeStruct((b, NC, H), f32),
        ],
    )(pos_r, pos, feat, cent, w1f, w1p, b1r, W2, b2r)

    return posq[:, :, :3], featq


# speedup vs baseline: 14.9048x; 1.9882x over previous
import functools

import jax
import jax.numpy as jnp
from jax import lax
from jax.experimental import pallas as pl

N = 2048
NC = 512
K = 16
F = 128
H = 256


def _fps_kernel(x_ref, y_ref, z_ref, out_ref):
    B = x_ref.shape[0]
    x = x_ref[...]
    y = y_ref[...]
    z = z_ref[...]
    iota = lax.broadcasted_iota(jnp.int32, (B, N), 1)
    islot = lax.broadcasted_iota(jnp.int32, (B, NC), 1)
    lx = x[:, 0:1]
    ly = y[:, 0:1]
    lz = z[:, 0:1]
    dmin0 = jnp.full((B, N), 1e10, jnp.float32)
    slots0 = jnp.zeros((B, NC), jnp.int32)

    def step(i, carry):
        dmin, lx, ly, lz, slots = carry
        d2 = (x - lx) ** 2 + (y - ly) ** 2 + (z - lz) ** 2
        dmin = jnp.minimum(dmin, d2)
        m = jnp.max(dmin, axis=1, keepdims=True)
        idx = jnp.min(jnp.where(dmin == m, iota, N), axis=1, keepdims=True)
        slots = jnp.where(islot == i, idx, slots)
        sel = iota == idx
        lx = jnp.sum(jnp.where(sel, x, 0.0), axis=1, keepdims=True)
        ly = jnp.sum(jnp.where(sel, y, 0.0), axis=1, keepdims=True)
        lz = jnp.sum(jnp.where(sel, z, 0.0), axis=1, keepdims=True)
        return dmin, lx, ly, lz, slots

    carry = lax.fori_loop(1, NC, step, (dmin0, lx, ly, lz, slots0))
    out_ref[...] = carry[4]


def _split(v):
    hi = v.astype(jnp.bfloat16)
    lo = (v - hi.astype(jnp.float32)).astype(jnp.bfloat16)
    return hi, lo


def _main_kernel(pos_r_ref, pos_c_ref, feat_ref, cent_ref,
                 w1f_ref, w1p_ref, b1_ref, w2_ref, b2_ref,
                 posq_ref, featq_ref):
    f32 = jnp.float32
    x = pos_r_ref[0, 0:1, :]
    y = pos_r_ref[0, 1:2, :]
    z = pos_r_ref[0, 2:3, :]
    xc = pos_c_ref[0, :, 0:1]
    yc = pos_c_ref[0, :, 1:2]
    zc = pos_c_ref[0, :, 2:3]
    feat = feat_ref[0]
    cent = cent_ref[0]

    bf16 = jnp.bfloat16
    P = (xc * w1p_ref[0:1, :] + yc * w1p_ref[1:2, :] + zc * w1p_ref[2:3, :])
    A = jnp.dot(feat, w1f_ref[...], preferred_element_type=f32) + P
    Ahi, Alo = _split(A)

    iota = lax.broadcasted_iota(jnp.int32, (NC, N), 1)
    ohc = (iota == cent)
    qx = jnp.sum(jnp.where(ohc, x, 0.0), axis=1, keepdims=True)
    qy = jnp.sum(jnp.where(ohc, y, 0.0), axis=1, keepdims=True)
    qz = jnp.sum(jnp.where(ohc, z, 0.0), axis=1, keepdims=True)
    ohcb = ohc.astype(bf16)
    Phi, Plo = _split(P)
    qproj = (jnp.dot(ohcb, Phi, preferred_element_type=f32)
             + jnp.dot(ohcb, Plo, preferred_element_type=f32))
    cadd = b1_ref[...] - qproj

    D = (qx - x) ** 2 + (qy - y) ** 2 + (qz - z) ** 2

    acc = jnp.full((NC, H), -jnp.inf, f32)
    w2hi, w2lo = _split(w2_ref[...])
    b2 = b2_ref[...]
    for _ in range(K):
        mn = jnp.min(D, axis=1, keepdims=True)
        idx = jnp.min(jnp.where(D == mn, iota, N), axis=1, keepdims=True)
        oh = (iota == idx)
        ohb = oh.astype(bf16)
        g = (jnp.dot(ohb, Ahi, preferred_element_type=f32)
             + jnp.dot(ohb, Alo, preferred_element_type=f32))
        h1 = jnp.maximum(g + cadd, 0.0)
        h1hi, h1lo = _split(h1)
        h2 = (jnp.dot(h1hi, w2hi, preferred_element_type=f32)
              + jnp.dot(h1hi, w2lo, preferred_element_type=f32)
              + jnp.dot(h1lo, w2hi, preferred_element_type=f32)) + b2
        h2 = jnp.maximum(h2, 0.0)
        acc = jnp.maximum(acc, h2)
        D = jnp.where(oh, jnp.inf, D)

    lane = lax.broadcasted_iota(jnp.int32, (NC, 128), 1)
    pq = jnp.where(lane == 0, qx, jnp.where(lane == 1, qy,
                                            jnp.where(lane == 2, qz, 0.0)))
    posq_ref[0] = pq
    featq_ref[0] = acc


@jax.jit
def kernel(feat, pos, W1, b1, W2, b2):
    b = feat.shape[0]
    f32 = jnp.float32
    pos_r = jnp.transpose(pos, (0, 2, 1))

    xb = pos_r[:, 0, :]
    yb = pos_r[:, 1, :]
    zb = pos_r[:, 2, :]
    cent = pl.pallas_call(
        _fps_kernel,
        out_shape=jax.ShapeDtypeStruct((b, NC), jnp.int32),
    )(xb, yb, zb)
    cent = cent.reshape(b, NC, 1)

    w1f = W1[:F]
    w1p = jnp.zeros((8, H), f32).at[:3].set(W1[F:])
    b1r = b1.reshape(1, H)
    b2r = b2.reshape(1, H)

    posq, featq = pl.pallas_call(
        _main_kernel,
        grid=(b,),
        in_specs=[
            pl.BlockSpec((1, 3, N), lambda i: (i, 0, 0)),
            pl.BlockSpec((1, N, 3), lambda i: (i, 0, 0)),
            pl.BlockSpec((1, N, F), lambda i: (i, 0, 0)),
            pl.BlockSpec((1, NC, 1), lambda i: (i, 0, 0)),
            pl.BlockSpec((F, H), lambda i: (0, 0)),
            pl.BlockSpec((8, H), lambda i: (0, 0)),
            pl.BlockSpec((1, H), lambda i: (0, 0)),
            pl.BlockSpec((H, H), lambda i: (0, 0)),
            pl.BlockSpec((1, H), lambda i: (0, 0)),
        ],
        out_specs=[
            pl.BlockSpec((1, NC, 128), lambda i: (i, 0, 0)),
            pl.BlockSpec((1, NC, H), lambda i: (i, 0, 0)),
        ],
        out_shape=[
            jax.ShapeDtypeStruct((b, NC, 128), f32),
            jax.ShapeDtypeStruct((b, NC, H), f32),
        ],
    )(pos_r, pos, feat, cent, w1f, w1p, b1r, W2, b2r)

    return posq[:, :, :3], featq


# speedup vs baseline: 18.0714x; 1.2124x over previous
import functools

import jax
import jax.numpy as jnp
from jax import lax
from jax.experimental import pallas as pl
from jax.experimental.pallas import tpu as pltpu
from jax.experimental.pallas import tpu_sc as plsc

N = 2048
NC = 512
K = 16
F = 128
H = 256

NWORK = 32
CHUNK = 128


def _fps_kernel(x_ref, y_ref, z_ref, out_ref):
    B = x_ref.shape[0]
    x = x_ref[...]
    y = y_ref[...]
    z = z_ref[...]
    iota = lax.broadcasted_iota(jnp.int32, (B, N), 1)
    islot = lax.broadcasted_iota(jnp.int32, (B, NC), 1)
    lx = x[:, 0:1]
    ly = y[:, 0:1]
    lz = z[:, 0:1]
    dmin0 = jnp.full((B, N), 1e10, jnp.float32)
    slots0 = jnp.zeros((B, NC), jnp.int32)

    def step(i, carry):
        dmin, lx, ly, lz, slots = carry
        d2 = (x - lx) ** 2 + (y - ly) ** 2 + (z - lz) ** 2
        dmin = jnp.minimum(dmin, d2)
        m = jnp.max(dmin, axis=1, keepdims=True)
        idx = jnp.min(jnp.where(dmin == m, iota, N), axis=1, keepdims=True)
        slots = jnp.where(islot == i, idx, slots)
        sel = iota == idx
        lx = jnp.sum(jnp.where(sel, x, 0.0), axis=1, keepdims=True)
        ly = jnp.sum(jnp.where(sel, y, 0.0), axis=1, keepdims=True)
        lz = jnp.sum(jnp.where(sel, z, 0.0), axis=1, keepdims=True)
        return dmin, lx, ly, lz, slots

    carry = lax.fori_loop(1, NC, step, (dmin0, lx, ly, lz, slots0))
    out_ref[...] = carry[4]


def _split(v):
    hi = v.astype(jnp.bfloat16)
    lo = (v - hi.astype(jnp.float32)).astype(jnp.bfloat16)
    return hi, lo


def _knn_kernel(pos_r_ref, pos_c_ref, feat_ref, cent_ref,
                w1f_ref, w1p_ref, b1_ref,
                posq_ref, a_ref, cadd_ref, nbr_ref):
    f32 = jnp.float32
    bf16 = jnp.bfloat16
    b = pl.program_id(0)
    x = pos_r_ref[0, 0:1, :]
    y = pos_r_ref[0, 1:2, :]
    z = pos_r_ref[0, 2:3, :]
    xc = pos_c_ref[0, :, 0:1]
    yc = pos_c_ref[0, :, 1:2]
    zc = pos_c_ref[0, :, 2:3]
    feat = feat_ref[0]
    cent = cent_ref[0]

    P = (xc * w1p_ref[0:1, :] + yc * w1p_ref[1:2, :] + zc * w1p_ref[2:3, :])
    A = jnp.dot(feat, w1f_ref[...], preferred_element_type=f32) + P
    a_ref[0] = A

    iota = lax.broadcasted_iota(jnp.int32, (NC, N), 1)
    ohc = (iota == cent)
    qx = jnp.sum(jnp.where(ohc, x, 0.0), axis=1, keepdims=True)
    qy = jnp.sum(jnp.where(ohc, y, 0.0), axis=1, keepdims=True)
    qz = jnp.sum(jnp.where(ohc, z, 0.0), axis=1, keepdims=True)
    ohcb = ohc.astype(bf16)
    Phi, Plo = _split(P)
    qproj = (jnp.dot(ohcb, Phi, preferred_element_type=f32)
             + jnp.dot(ohcb, Plo, preferred_element_type=f32))
    cadd_ref[0] = b1_ref[...] - qproj

    D = (qx - x) ** 2 + (qy - y) ** 2 + (qz - z) ** 2

    for j in range(K):
        mn = jnp.min(D, axis=1, keepdims=True)
        idx = jnp.min(jnp.where(D == mn, iota, N), axis=1, keepdims=True)
        nbr_ref[0, j] = idx + b * N
        D = jnp.where(iota == idx, jnp.inf, D)

    lane = lax.broadcasted_iota(jnp.int32, (NC, 128), 1)
    pq = jnp.where(lane == 0, qx, jnp.where(lane == 1, qy,
                                            jnp.where(lane == 2, qz, 0.0)))
    posq_ref[0] = pq


def _sc_gather_body(table_hbm, idx_hbm, out_hbm, idx_v, rows_v, sem):
    wid = lax.axis_index("s") * 2 + lax.axis_index("c")
    nrows = idx_hbm.shape[0]
    per_w = nrows // NWORK
    base = wid * per_w

    def chunk(c, carry):
        off = base + c * CHUNK
        pltpu.sync_copy(idx_hbm.at[pl.ds(off, CHUNK)], idx_v)
        pltpu.async_copy(table_hbm.at[idx_v], rows_v, sem).wait()
        pltpu.sync_copy(rows_v, out_hbm.at[pl.ds(off, CHUNK)])
        return carry

    lax.fori_loop(0, per_w // CHUNK, chunk, 0)


def _sc_gather(table, idx):
    nrows = idx.shape[0]
    f = functools.partial(
        pl.kernel,
        mesh=plsc.VectorSubcoreMesh(core_axis_name="c", subcore_axis_name="s"),
        out_type=jax.ShapeDtypeStruct((nrows, H), jnp.float32),
        scratch_types=[
            pltpu.VMEM((CHUNK,), jnp.int32),
            pltpu.VMEM((CHUNK, H), jnp.float32),
            pltpu.SemaphoreType.DMA,
        ],
    )(_sc_gather_body)
    return f(table, idx)


def _mlp_kernel(g_ref, cadd_ref, w2hi_ref, w2lo_ref, b2_ref, out_ref):
    j = pl.program_id(1)
    h1 = jnp.maximum(g_ref[0, 0] + cadd_ref[0], 0.0)
    h1hi, h1lo = _split(h1)
    f32 = jnp.float32
    h2 = (jnp.dot(h1hi, w2hi_ref[...], preferred_element_type=f32)
          + jnp.dot(h1hi, w2lo_ref[...], preferred_element_type=f32)
          + jnp.dot(h1lo, w2hi_ref[...], preferred_element_type=f32)) + b2_ref[...]
    h2 = jnp.maximum(h2, 0.0)

    @pl.when(j == 0)
    def _():
        out_ref[0] = h2

    @pl.when(j > 0)
    def _():
        out_ref[0] = jnp.maximum(out_ref[0], h2)


@jax.jit
def kernel(feat, pos, W1, b1, W2, b2):
    b = feat.shape[0]
    f32 = jnp.float32
    pos_r = jnp.transpose(pos, (0, 2, 1))

    xb = pos_r[:, 0, :]
    yb = pos_r[:, 1, :]
    zb = pos_r[:, 2, :]
    cent = pl.pallas_call(
        _fps_kernel,
        out_shape=jax.ShapeDtypeStruct((b, NC), jnp.int32),
    )(xb, yb, zb)
    cent = cent.reshape(b, NC, 1)

    w1f = W1[:F]
    w1p = jnp.zeros((8, H), f32).at[:3].set(W1[F:])
    b1r = b1.reshape(1, H)
    b2r = b2.reshape(1, H)

    posq, A, cadd, nbr = pl.pallas_call(
        _knn_kernel,
        grid=(b,),
        in_specs=[
            pl.BlockSpec((1, 3, N), lambda i: (i, 0, 0)),
            pl.BlockSpec((1, N, 3), lambda i: (i, 0, 0)),
            pl.BlockSpec((1, N, F), lambda i: (i, 0, 0)),
            pl.BlockSpec((1, NC, 1), lambda i: (i, 0, 0)),
            pl.BlockSpec((F, H), lambda i: (0, 0)),
            pl.BlockSpec((8, H), lambda i: (0, 0)),
            pl.BlockSpec((1, H), lambda i: (0, 0)),
        ],
        out_specs=[
            pl.BlockSpec((1, NC, 128), lambda i: (i, 0, 0)),
            pl.BlockSpec((1, N, H), lambda i: (i, 0, 0)),
            pl.BlockSpec((1, NC, H), lambda i: (i, 0, 0)),
            pl.BlockSpec((1, K, NC, 1), lambda i: (i, 0, 0, 0)),
        ],
        out_shape=[
            jax.ShapeDtypeStruct((b, NC, 128), f32),
            jax.ShapeDtypeStruct((b, N, H), f32),
            jax.ShapeDtypeStruct((b, NC, H), f32),
            jax.ShapeDtypeStruct((b, K, NC, 1), jnp.int32),
        ],
    )(pos_r, pos, feat, cent, w1f, w1p, b1r)

    g = _sc_gather(A.reshape(b * N, H), nbr.reshape(b * K * NC))
    g = g.reshape(b, K, NC, H)

    w2hi = W2.astype(jnp.bfloat16)
    w2lo = (W2 - w2hi.astype(f32)).astype(jnp.bfloat16)

    featq = pl.pallas_call(
        _mlp_kernel,
        grid=(b, K),
        in_specs=[
            pl.BlockSpec((1, 1, NC, H), lambda i, j: (i, j, 0, 0)),
            pl.BlockSpec((1, NC, H), lambda i, j: (i, 0, 0)),
            pl.BlockSpec((H, H), lambda i, j: (0, 0)),
            pl.BlockSpec((H, H), lambda i, j: (0, 0)),
            pl.BlockSpec((1, H), lambda i, j: (0, 0)),
        ],
        out_specs=pl.BlockSpec((1, NC, H), lambda i, j: (i, 0, 0)),
        out_shape=jax.ShapeDtypeStruct((b, NC, H), f32),
    )(g, cadd, w2hi, w2lo, b2r)

    return posq[:, :, :3], featq
